# baseline (speedup 1.0000x reference)
import functools

import jax
import jax.numpy as jnp
from jax import lax
from jax.experimental import pallas as pl
from jax.experimental.pallas import tpu as pltpu
from jax.experimental.pallas import tpu_sc as plsc

NC = 2
NS = 16
NW = NC * NS
LK = 16
K = 80

F32 = jnp.float32


def _wid(cid, sid):
    return cid * NS + sid


def _make_sc_deg(n_nodes, chunks):
    mesh = plsc.VectorSubcoreMesh(core_axis_name="c", subcore_axis_name="s")

    @functools.partial(
        pl.kernel,
        out_type=jax.ShapeDtypeStruct((NW, n_nodes), F32),
        mesh=mesh,
        compiler_params=pltpu.CompilerParams(needs_layout_passes=False),
        scratch_types=[
            pltpu.VMEM((chunks, K), jnp.int32),
            pltpu.VMEM((n_nodes,), F32),
        ],
    )
    def sc_deg(row_hbm, deg_out, rbuf, degbuf):
        cid = lax.axis_index("c")
        sid = lax.axis_index("s")
        wid = _wid(cid, sid)
        pltpu.sync_copy(row_hbm.at[wid], rbuf)

        zero16 = jnp.zeros((LK,), F32)
        ones16 = jnp.ones((LK,), F32)

        def zero_body(i, _):
            degbuf[pl.ds(i * LK, LK)] = zero16
            return 0

        lax.fori_loop(0, n_nodes // LK, zero_body, 0)

        def chunk_body(c, _):
            def lane_body(j, _):
                idx = rbuf[c, pl.ds(j * LK, LK)]
                plsc.addupdate_scatter(degbuf, [idx], ones16)
                return 0

            lax.fori_loop(0, K // LK, lane_body, 0)
            return 0

        lax.fori_loop(0, chunks, chunk_body, 0)
        pltpu.sync_copy(degbuf, deg_out.at[wid])

    return sc_deg


def _make_sc_spmm_b(n_nodes, chunks, d, kk):
    mesh = plsc.VectorSubcoreMesh(core_axis_name="c", subcore_axis_name="s")
    npt = n_nodes // NS
    assert npt % kk == 0 and chunks % 2 == 0
    half = chunks // 2
    assert half % 2 == 0 and half % 8 == 0

    @functools.partial(
        pl.kernel,
        out_type=jax.ShapeDtypeStruct((NC, n_nodes, d), F32),
        mesh=mesh,
        compiler_params=pltpu.CompilerParams(needs_layout_passes=False),
        scratch_types=[
            pltpu.VMEM((half, kk), jnp.int32),
            pltpu.VMEM((half, kk), jnp.int32),
            pltpu.VMEM((kk, d), F32),
            pltpu.VMEM((kk, d), F32),
            pltpu.VMEM_SHARED((n_nodes, d), F32),
            pltpu.SemaphoreType.DMA,
            pltpu.SemaphoreType.DMA,
        ],
    )
    def sc_spmm(row_hbm, col_hbm, xs_hbm, zt_out,
                rbuf, cbuf, gbuf0, gbuf1, zsh, sem0, sem1):
        cid = lax.axis_index("c")
        sid = lax.axis_index("s")
        wid = _wid(cid, sid)

        zero16 = jnp.zeros((LK,), F32)

        def zero_zb(i, _):
            def zlane(j, _):
                gbuf0[i, pl.ds(j * LK, LK)] = zero16
                return 0
            lax.fori_loop(0, d // LK, zlane, 0)
            return 0

        lax.fori_loop(0, kk, zero_zb, 0)

        def zero_stripe(k_, _):
            pltpu.sync_copy(gbuf0, zsh.at[pl.ds(sid * npt + k_ * kk, kk)])
            return 0

        lax.fori_loop(0, npt // kk, zero_stripe, 0)
        plsc.subcore_barrier()

        def g_issue(c, buf, sem_):
            pltpu.async_copy(xs_hbm.at[cbuf.at[c]], buf, sem_)

        def g_wait(c, buf, sem_):
            pltpu.make_async_copy(xs_hbm.at[cbuf.at[c]], buf, sem_).wait()

        def s_add(c, buf):
            pltpu.sync_copy(buf, zsh.at[rbuf.at[c]], add=True)

        def do_half(hf, _):
            pltpu.sync_copy(row_hbm.at[wid].at[pl.ds(hf * half, half)], rbuf)
            pltpu.sync_copy(col_hbm.at[wid].at[pl.ds(hf * half, half)], cbuf)
            g_issue(0, gbuf0, sem0)

            def pair(i, _):
                c0 = 2 * i
                g_issue(c0 + 1, gbuf1, sem1)
                g_wait(c0, gbuf0, sem0)
                s_add(c0, gbuf0)

                @pl.when(c0 + 2 < half)
                def _():
                    g_issue(c0 + 2, gbuf0, sem0)

                g_wait(c0 + 1, gbuf1, sem1)
                s_add(c0 + 1, gbuf1)
                return 0

            lax.fori_loop(0, half // 2, pair, 0)
            return 0

        lax.fori_loop(0, 2, do_half, 0)
        plsc.subcore_barrier()

        pltpu.sync_copy(zsh.at[pl.ds(sid * npt, npt)],
                        zt_out.at[cid].at[pl.ds(sid * npt, npt)])

    return sc_spmm


def _make_sc_st(n_nodes, chunks):
    mesh = plsc.VectorSubcoreMesh(core_axis_name="c", subcore_axis_name="s")

    @functools.partial(
        pl.kernel,
        out_type=jax.ShapeDtypeStruct((NW, n_nodes), F32),
        mesh=mesh,
        compiler_params=pltpu.CompilerParams(needs_layout_passes=False),
        scratch_types=[
            pltpu.VMEM((chunks, K), jnp.int32),
            pltpu.VMEM((chunks, K), jnp.int32),
            pltpu.VMEM((n_nodes,), F32),
            pltpu.VMEM((n_nodes,), F32),
        ],
    )
    def sc_st(row_hbm, col_hbm, dis_hbm, st_out, rbuf, cbuf, disbuf, sbuf):
        cid = lax.axis_index("c")
        sid = lax.axis_index("s")
        wid = _wid(cid, sid)
        pltpu.sync_copy(row_hbm.at[wid], rbuf)
        pltpu.sync_copy(col_hbm.at[wid], cbuf)
        pltpu.sync_copy(dis_hbm, disbuf)

        zero16 = jnp.zeros((LK,), F32)

        def zero_s(i, _):
            sbuf[pl.ds(i * LK, LK)] = zero16
            return 0

        lax.fori_loop(0, n_nodes // LK, zero_s, 0)

        def chunk_body(c, _):
            def lane_body(j, _):
                cidx = cbuf[c, pl.ds(j * LK, LK)]
                vals = plsc.load_gather(disbuf, [cidx])
                ridx = rbuf[c, pl.ds(j * LK, LK)]
                plsc.addupdate_scatter(sbuf, [ridx], vals)
                return 0

            lax.fori_loop(0, K // LK, lane_body, 0)
            return 0

        lax.fori_loop(0, chunks, chunk_body, 0)
        pltpu.sync_copy(sbuf, st_out.at[wid])

    return sc_st


def _make_sc_spmm_c(n_nodes, chunks, d, slabs, kk):
    mesh = plsc.VectorSubcoreMesh(core_axis_name="c", subcore_axis_name="s")
    npt = n_nodes // NS
    assert npt % kk == 0 and chunks % 2 == 0
    half = chunks // 2
    assert half % 2 == 0 and half % 8 == 0

    @functools.partial(
        pl.kernel,
        out_type=jax.ShapeDtypeStruct((NC, slabs, n_nodes, d), F32),
        mesh=mesh,
        compiler_params=pltpu.CompilerParams(needs_layout_passes=False),
        scratch_types=[
            pltpu.VMEM((half, kk), jnp.int32),
            pltpu.VMEM((half, kk), jnp.int32),
            pltpu.VMEM((kk, d), F32),
            pltpu.VMEM((kk, d), F32),
            pltpu.VMEM_SHARED((n_nodes, d), F32),
            pltpu.SemaphoreType.DMA,
            pltpu.SemaphoreType.DMA,
        ],
    )
    def sc_spmm_c(row_hbm, col_hbm, y_hbm, o_out,
                  rbuf, cbuf, gbuf0, gbuf1, zsh, sem0, sem1):
        cid = lax.axis_index("c")
        sid = lax.axis_index("s")
        wid = _wid(cid, sid)

        zero16 = jnp.zeros((LK,), F32)

        def slab_body(s, _):
            ysl = y_hbm.at[s]

            def zero_zb(i, _):
                def zlane(j, _):
                    gbuf0[i, pl.ds(j * LK, LK)] = zero16
                    return 0
                lax.fori_loop(0, d // LK, zlane, 0)
                return 0

            lax.fori_loop(0, kk, zero_zb, 0)

            def zero_stripe(k_, _):
                pltpu.sync_copy(gbuf0, zsh.at[pl.ds(sid * npt + k_ * kk, kk)])
                return 0

            lax.fori_loop(0, npt // kk, zero_stripe, 0)
            plsc.subcore_barrier()

            def g_issue(c, buf, sem_):
                pltpu.async_copy(ysl.at[cbuf.at[c]], buf, sem_)

            def g_wait(c, buf, sem_):
                pltpu.make_async_copy(ysl.at[cbuf.at[c]], buf, sem_).wait()

            def s_add(c, buf):
                pltpu.sync_copy(buf, zsh.at[rbuf.at[c]], add=True)

            def do_half(hf, _):
                pltpu.sync_copy(row_hbm.at[wid].at[pl.ds(hf * half, half)], rbuf)
                pltpu.sync_copy(col_hbm.at[wid].at[pl.ds(hf * half, half)], cbuf)
                g_issue(0, gbuf0, sem0)

                def pair(i, _):
                    c0 = 2 * i
                    g_issue(c0 + 1, gbuf1, sem1)
                    g_wait(c0, gbuf0, sem0)
                    s_add(c0, gbuf0)

                    @pl.when(c0 + 2 < half)
                    def _():
                        g_issue(c0 + 2, gbuf0, sem0)

                    g_wait(c0 + 1, gbuf1, sem1)
                    s_add(c0 + 1, gbuf1)
                    return 0

                lax.fori_loop(0, half // 2, pair, 0)
                return 0

            lax.fori_loop(0, 2, do_half, 0)
            plsc.subcore_barrier()
            pltpu.sync_copy(zsh.at[pl.ds(sid * npt, npt)],
                            o_out.at[cid].at[s].at[pl.ds(sid * npt, npt)])
            plsc.subcore_barrier()
            return 0

        lax.fori_loop(0, slabs, slab_body, 0)

    return sc_spmm_c


def _tc1_body(degp_ref, x_ref, dis_ref, xs_ref):
    n, d = x_ref.shape
    npad = xs_ref.shape[0]
    deg = jnp.sum(degp_ref[...], axis=0)
    dis = jnp.where(deg > 0.0, lax.rsqrt(deg), 0.0)
    dis_ref[...] = dis[:, None]
    xs_ref[0:n, :] = dis[0:n, None] * x_ref[...]
    xs_ref[n:npad, :] = jnp.zeros((npad - n, d), F32)


def _tc2_body(zt_ref, st_ref, dis_ref, w1_ref, b1_ref, w2_ref, b2_ref, y_ref):
    nb = dis_ref.shape[0]
    i = pl.program_id(0)
    dis = dis_ref[...]
    z = (zt_ref[0] + zt_ref[1]) * dis
    st = st_ref[:, pl.ds(i * nb, nb)]
    s = jnp.sum(st, axis=0)[:, None] * dis
    h1 = jnp.maximum(
        jnp.dot(z, w1_ref[...], preferred_element_type=F32)
        + s * b1_ref[...], 0.0)
    slabs = y_ref.shape[0]
    d = y_ref.shape[2]
    for k in range(slabs):
        yk = (jnp.dot(h1, w2_ref[:, k * d:(k + 1) * d],
                      preferred_element_type=F32)
              + b2_ref[:, k * d:(k + 1) * d]) * dis
        y_ref[k] = yk


def _tc3_body(o_ref, dis_ref, out_ref):
    dis = dis_ref[...]
    acc = o_ref[0] + o_ref[1]
    slabs = acc.shape[0]
    d = acc.shape[2]
    for k in range(slabs):
        out_ref[:, k * d:(k + 1) * d] = jnp.maximum(acc[k] * dis, 0.0)


def kernel(x, edge_index, W1, b1, W2, b2):
    n, d = x.shape
    h = W1.shape[1]
    e = edge_index.shape[1]
    slabs = h // d
    chunks = e // (NW * K)
    assert e == NW * chunks * K
    npad = -(-n // (NS * 128)) * (NS * 128)
    assert n % 8 == 0

    kk = 128
    epad = -(-e // (NW * kk * 4)) * (NW * kk * 4)
    chunks2 = epad // (NW * kk)
    pad_edge = jnp.full((epad - e,), npad - 1, jnp.int32)

    row = edge_index[0].reshape(NW, chunks, K)
    col = edge_index[1].reshape(NW, chunks, K)
    row2 = jnp.concatenate([edge_index[0], pad_edge]).reshape(NW, chunks2, kk)
    col2 = jnp.concatenate([edge_index[1], pad_edge]).reshape(NW, chunks2, kk)

    degp = _make_sc_deg(npad, chunks)(row)

    dis2, xs = pl.pallas_call(
        _tc1_body,
        out_shape=[
            jax.ShapeDtypeStruct((npad, 1), F32),
            jax.ShapeDtypeStruct((npad, d), F32),
        ],
    )(degp, x)

    zt = _make_sc_spmm_b(npad, chunks2, d, kk)(row2, col2, xs)
    st = _make_sc_st(npad, chunks)(row, col, dis2.reshape(npad))

    nb = 1024
    grid = npad // nb
    y = pl.pallas_call(
        _tc2_body,
        grid=(grid,),
        in_specs=[
            pl.BlockSpec((NC, nb, d), lambda i: (0, i, 0)),
            pl.BlockSpec((NW, npad), lambda i: (0, 0)),
            pl.BlockSpec((nb, 1), lambda i: (i, 0)),
            pl.BlockSpec((d, h), lambda i: (0, 0)),
            pl.BlockSpec((1, h), lambda i: (0, 0)),
            pl.BlockSpec((h, h), lambda i: (0, 0)),
            pl.BlockSpec((1, h), lambda i: (0, 0)),
        ],
        out_specs=pl.BlockSpec((slabs, nb, d), lambda i: (0, i, 0)),
        out_shape=jax.ShapeDtypeStruct((slabs, npad, d), F32),
    )(zt, st, dis2, W1, b1.reshape(1, h), W2, b2.reshape(1, h))

    o = _make_sc_spmm_c(npad, chunks2, d, slabs, kk)(row2, col2, y)

    nb3 = 1000
    grid3 = n // nb3
    out = pl.pallas_call(
        _tc3_body,
        grid=(grid3,),
        in_specs=[
            pl.BlockSpec((NC, slabs, nb3, d), lambda i: (0, 0, i, 0)),
            pl.BlockSpec((nb3, 1), lambda i: (i, 0)),
        ],
        out_specs=pl.BlockSpec((nb3, h), lambda i: (i, 0)),
        out_shape=jax.ShapeDtypeStruct((n, h), F32),
    )(o, dis2)

    return out


# speedup vs baseline: 3.7368x; 3.7368x over previous
import functools

import jax
import jax.numpy as jnp
from jax import lax
from jax.experimental import pallas as pl
from jax.experimental.pallas import tpu as pltpu
from jax.experimental.pallas import tpu_sc as plsc

NC = 2
NS = 16
NW = NC * NS
LK = 16
K = 80

F32 = jnp.float32


def _wid(cid, sid):
    return cid * NS + sid


def _make_sc_deg(n_nodes, chunks):
    mesh = plsc.VectorSubcoreMesh(core_axis_name="c", subcore_axis_name="s")

    @functools.partial(
        pl.kernel,
        out_type=jax.ShapeDtypeStruct((NW, n_nodes), F32),
        mesh=mesh,
        compiler_params=pltpu.CompilerParams(needs_layout_passes=False),
        scratch_types=[
            pltpu.VMEM((chunks, K), jnp.int32),
            pltpu.VMEM((n_nodes,), F32),
        ],
    )
    def sc_deg(row_hbm, deg_out, rbuf, degbuf):
        cid = lax.axis_index("c")
        sid = lax.axis_index("s")
        wid = _wid(cid, sid)
        pltpu.sync_copy(row_hbm.at[wid], rbuf)

        zero16 = jnp.zeros((LK,), F32)
        ones16 = jnp.ones((LK,), F32)

        def zero_body(i, _):
            degbuf[pl.ds(i * LK, LK)] = zero16
            return 0

        lax.fori_loop(0, n_nodes // LK, zero_body, 0)

        def chunk_body(c, _):
            def lane_body(j, _):
                idx = rbuf[c, pl.ds(j * LK, LK)]
                plsc.addupdate_scatter(degbuf, [idx], ones16)
                return 0

            lax.fori_loop(0, K // LK, lane_body, 0)
            return 0

        lax.fori_loop(0, chunks, chunk_body, 0)
        pltpu.sync_copy(degbuf, deg_out.at[wid])

    return sc_deg


def _make_sc_spmm_b(n_nodes, chunks, d, kk):
    mesh = plsc.VectorSubcoreMesh(core_axis_name="c", subcore_axis_name="s")
    npt = n_nodes // NS
    assert npt % kk == 0 and chunks % 2 == 0
    half = chunks // 2
    assert half % 2 == 0 and half % 8 == 0

    @functools.partial(
        pl.kernel,
        out_type=jax.ShapeDtypeStruct((NC, n_nodes, d), F32),
        mesh=mesh,
        compiler_params=pltpu.CompilerParams(needs_layout_passes=False),
        scratch_types=[
            pltpu.VMEM((half, kk), jnp.int32),
            pltpu.VMEM((half, kk), jnp.int32),
            pltpu.VMEM((kk, d), F32),
            pltpu.VMEM((kk, d), F32),
            pltpu.VMEM_SHARED((n_nodes, d), F32),
            pltpu.SemaphoreType.DMA,
            pltpu.SemaphoreType.DMA,
        ],
    )
    def sc_spmm(row_hbm, col_hbm, xs_hbm, zt_out,
                rbuf, cbuf, gbuf0, gbuf1, zsh, sem0, sem1):
        cid = lax.axis_index("c")
        sid = lax.axis_index("s")
        wid = _wid(cid, sid)

        zero16 = jnp.zeros((LK,), F32)

        def zero_zb(i, _):
            def zlane(j, _):
                gbuf0[i, pl.ds(j * LK, LK)] = zero16
                return 0
            lax.fori_loop(0, d // LK, zlane, 0)
            return 0

        lax.fori_loop(0, kk, zero_zb, 0)

        def zero_stripe(k_, _):
            pltpu.sync_copy(gbuf0, zsh.at[pl.ds(sid * npt + k_ * kk, kk)])
            return 0

        lax.fori_loop(0, npt // kk, zero_stripe, 0)
        plsc.subcore_barrier()

        def g_issue(c, buf, sem_):
            pltpu.async_copy(xs_hbm.at[cbuf.at[c]], buf, sem_)

        def g_wait(c, buf, sem_):
            pltpu.make_async_copy(xs_hbm.at[cbuf.at[c]], buf, sem_).wait()

        def s_add(c, buf):
            pltpu.sync_copy(buf, zsh.at[rbuf.at[c]], add=True)

        def do_half(hf, _):
            pltpu.sync_copy(row_hbm.at[wid].at[pl.ds(hf * half, half)], rbuf)
            pltpu.sync_copy(col_hbm.at[wid].at[pl.ds(hf * half, half)], cbuf)
            g_issue(0, gbuf0, sem0)

            def pair(i, _):
                c0 = 2 * i
                g_issue(c0 + 1, gbuf1, sem1)
                g_wait(c0, gbuf0, sem0)
                s_add(c0, gbuf0)

                @pl.when(c0 + 2 < half)
                def _():
                    g_issue(c0 + 2, gbuf0, sem0)

                g_wait(c0 + 1, gbuf1, sem1)
                s_add(c0 + 1, gbuf1)
                return 0

            lax.fori_loop(0, half // 2, pair, 0)
            return 0

        lax.fori_loop(0, 2, do_half, 0)
        plsc.subcore_barrier()

        pltpu.sync_copy(zsh.at[pl.ds(sid * npt, npt)],
                        zt_out.at[cid].at[pl.ds(sid * npt, npt)])

    return sc_spmm


def _make_sc_st(n_nodes, chunks):
    mesh = plsc.VectorSubcoreMesh(core_axis_name="c", subcore_axis_name="s")

    @functools.partial(
        pl.kernel,
        out_type=jax.ShapeDtypeStruct((NW, n_nodes), F32),
        mesh=mesh,
        compiler_params=pltpu.CompilerParams(needs_layout_passes=False),
        scratch_types=[
            pltpu.VMEM((chunks, K), jnp.int32),
            pltpu.VMEM((chunks, K), jnp.int32),
            pltpu.VMEM((n_nodes,), F32),
            pltpu.VMEM((n_nodes,), F32),
        ],
    )
    def sc_st(row_hbm, col_hbm, dis_hbm, st_out, rbuf, cbuf, disbuf, sbuf):
        cid = lax.axis_index("c")
        sid = lax.axis_index("s")
        wid = _wid(cid, sid)
        pltpu.sync_copy(row_hbm.at[wid], rbuf)
        pltpu.sync_copy(col_hbm.at[wid], cbuf)
        pltpu.sync_copy(dis_hbm, disbuf)

        zero16 = jnp.zeros((LK,), F32)

        def zero_s(i, _):
            sbuf[pl.ds(i * LK, LK)] = zero16
            return 0

        lax.fori_loop(0, n_nodes // LK, zero_s, 0)

        def chunk_body(c, _):
            def lane_body(j, _):
                cidx = cbuf[c, pl.ds(j * LK, LK)]
                vals = plsc.load_gather(disbuf, [cidx])
                ridx = rbuf[c, pl.ds(j * LK, LK)]
                plsc.addupdate_scatter(sbuf, [ridx], vals)
                return 0

            lax.fori_loop(0, K // LK, lane_body, 0)
            return 0

        lax.fori_loop(0, chunks, chunk_body, 0)
        pltpu.sync_copy(sbuf, st_out.at[wid])

    return sc_st


def _make_sc_spmm_c(n_nodes, chunks, d, slabs, kk):
    mesh = plsc.VectorSubcoreMesh(core_axis_name="c", subcore_axis_name="s")
    npt = n_nodes // NS
    assert npt % kk == 0 and chunks % 2 == 0
    half = chunks // 2
    assert half % 2 == 0 and half % 8 == 0

    @functools.partial(
        pl.kernel,
        out_type=jax.ShapeDtypeStruct((NC, slabs, n_nodes, d), F32),
        mesh=mesh,
        compiler_params=pltpu.CompilerParams(needs_layout_passes=False),
        scratch_types=[
            pltpu.VMEM((half, kk), jnp.int32),
            pltpu.VMEM((half, kk), jnp.int32),
            pltpu.VMEM((kk, d), F32),
            pltpu.VMEM((kk, d), F32),
            pltpu.VMEM_SHARED((n_nodes, d), F32),
            pltpu.SemaphoreType.DMA,
            pltpu.SemaphoreType.DMA,
        ],
    )
    def sc_spmm_c(row_hbm, col_hbm, y_hbm, o_out,
                  rbuf, cbuf, gbuf0, gbuf1, zsh, sem0, sem1):
        cid = lax.axis_index("c")
        sid = lax.axis_index("s")
        wid = _wid(cid, sid)

        zero16 = jnp.zeros((LK,), F32)

        def slab_body(s, _):
            ysl = y_hbm.at[s]

            def zero_zb(i, _):
                def zlane(j, _):
                    gbuf0[i, pl.ds(j * LK, LK)] = zero16
                    return 0
                lax.fori_loop(0, d // LK, zlane, 0)
                return 0

            lax.fori_loop(0, kk, zero_zb, 0)

            def zero_stripe(k_, _):
                pltpu.sync_copy(gbuf0, zsh.at[pl.ds(sid * npt + k_ * kk, kk)])
                return 0

            lax.fori_loop(0, npt // kk, zero_stripe, 0)
            plsc.subcore_barrier()

            def g_issue(c, buf, sem_):
                pltpu.async_copy(ysl.at[cbuf.at[c]], buf, sem_)

            def g_wait(c, buf, sem_):
                pltpu.make_async_copy(ysl.at[cbuf.at[c]], buf, sem_).wait()

            def s_add(c, buf):
                pltpu.sync_copy(buf, zsh.at[rbuf.at[c]], add=True)

            def do_half(hf, _):
                pltpu.sync_copy(row_hbm.at[wid].at[pl.ds(hf * half, half)], rbuf)
                pltpu.sync_copy(col_hbm.at[wid].at[pl.ds(hf * half, half)], cbuf)
                g_issue(0, gbuf0, sem0)

                def pair(i, _):
                    c0 = 2 * i
                    g_issue(c0 + 1, gbuf1, sem1)
                    g_wait(c0, gbuf0, sem0)
                    s_add(c0, gbuf0)

                    @pl.when(c0 + 2 < half)
                    def _():
                        g_issue(c0 + 2, gbuf0, sem0)

                    g_wait(c0 + 1, gbuf1, sem1)
                    s_add(c0 + 1, gbuf1)
                    return 0

                lax.fori_loop(0, half // 2, pair, 0)
                return 0

            lax.fori_loop(0, 2, do_half, 0)
            plsc.subcore_barrier()
            pltpu.sync_copy(zsh.at[pl.ds(sid * npt, npt)],
                            o_out.at[cid].at[s].at[pl.ds(sid * npt, npt)])
            plsc.subcore_barrier()
            return 0

        lax.fori_loop(0, slabs, slab_body, 0)

    return sc_spmm_c


def _tc1_body(degp_ref, x_ref, dis_ref, xs_ref):
    n, d = x_ref.shape
    npad = xs_ref.shape[0]
    deg = jnp.sum(degp_ref[...], axis=0)
    dis = jnp.where(deg > 0.0, lax.rsqrt(deg), 0.0)
    dis_ref[...] = dis[:, None]
    xs_ref[0:n, :] = dis[0:n, None] * x_ref[...]
    xs_ref[n:npad, :] = jnp.zeros((npad - n, d), F32)


def _tc2_body(zt_ref, st_ref, dis_ref, w1_ref, b1_ref, w2_ref, b2_ref, y_ref):
    nb = dis_ref.shape[0]
    i = pl.program_id(0)
    dis = dis_ref[...]
    z = (zt_ref[0] + zt_ref[1]) * dis
    st = st_ref[:, pl.ds(i * nb, nb)]
    s = jnp.sum(st, axis=0)[:, None] * dis
    h1 = jnp.maximum(
        jnp.dot(z, w1_ref[...], preferred_element_type=F32)
        + s * b1_ref[...], 0.0)
    slabs = y_ref.shape[0]
    d = y_ref.shape[2]
    for k in range(slabs):
        yk = (jnp.dot(h1, w2_ref[:, k * d:(k + 1) * d],
                      preferred_element_type=F32)
              + b2_ref[:, k * d:(k + 1) * d]) * dis
        y_ref[k] = yk


def _tc3_body(o_ref, dis_ref, out_ref):
    dis = dis_ref[...]
    acc = o_ref[0] + o_ref[1]
    slabs = acc.shape[0]
    d = acc.shape[2]
    for k in range(slabs):
        out_ref[:, k * d:(k + 1) * d] = jnp.maximum(acc[k] * dis, 0.0)


def kernel(x, edge_index, W1, b1, W2, b2):
    n, d = x.shape
    h = W1.shape[1]
    e = edge_index.shape[1]
    slabs = h // d
    chunks = e // (NW * K)
    assert e == NW * chunks * K
    npad = -(-n // (NS * 128)) * (NS * 128)
    assert n % 8 == 0

    kk = 128
    epad = -(-e // (NW * kk * 4)) * (NW * kk * 4)
    chunks2 = epad // (NW * kk)
    assert npad > n or epad == e
    pad_edge = n + (jnp.arange(epad - e, dtype=jnp.int32) % (npad - n))

    row = edge_index[0].reshape(NW, chunks, K)
    col = edge_index[1].reshape(NW, chunks, K)
    row2 = jnp.concatenate([edge_index[0], pad_edge]).reshape(NW, chunks2, kk)
    col2 = jnp.concatenate([edge_index[1], pad_edge]).reshape(NW, chunks2, kk)

    degp = _make_sc_deg(npad, chunks)(row)

    dis2, xs = pl.pallas_call(
        _tc1_body,
        out_shape=[
            jax.ShapeDtypeStruct((npad, 1), F32),
            jax.ShapeDtypeStruct((npad, d), F32),
        ],
    )(degp, x)

    zt = _make_sc_spmm_b(npad, chunks2, d, kk)(row2, col2, xs)
    st = _make_sc_st(npad, chunks)(row, col, dis2.reshape(npad))

    nb = 1024
    grid = npad // nb
    y = pl.pallas_call(
        _tc2_body,
        grid=(grid,),
        in_specs=[
            pl.BlockSpec((NC, nb, d), lambda i: (0, i, 0)),
            pl.BlockSpec((NW, npad), lambda i: (0, 0)),
            pl.BlockSpec((nb, 1), lambda i: (i, 0)),
            pl.BlockSpec((d, h), lambda i: (0, 0)),
            pl.BlockSpec((1, h), lambda i: (0, 0)),
            pl.BlockSpec((h, h), lambda i: (0, 0)),
            pl.BlockSpec((1, h), lambda i: (0, 0)),
        ],
        out_specs=pl.BlockSpec((slabs, nb, d), lambda i: (0, i, 0)),
        out_shape=jax.ShapeDtypeStruct((slabs, npad, d), F32),
    )(zt, st, dis2, W1, b1.reshape(1, h), W2, b2.reshape(1, h))

    o = _make_sc_spmm_c(npad, chunks2, d, slabs, kk)(row2, col2, y)

    nb3 = 1000
    grid3 = n // nb3
    out = pl.pallas_call(
        _tc3_body,
        grid=(grid3,),
        in_specs=[
            pl.BlockSpec((NC, slabs, nb3, d), lambda i: (0, 0, i, 0)),
            pl.BlockSpec((nb3, 1), lambda i: (i, 0)),
        ],
        out_specs=pl.BlockSpec((nb3, h), lambda i: (i, 0)),
        out_shape=jax.ShapeDtypeStruct((n, h), F32),
    )(o, dis2)

    return out


# speedup vs baseline: 3.8289x; 1.0247x over previous
import functools

import jax
import jax.numpy as jnp
from jax import lax
from jax.experimental import pallas as pl
from jax.experimental.pallas import tpu as pltpu
from jax.experimental.pallas import tpu_sc as plsc

NC = 2
NS = 16
NW = NC * NS
LK = 16
K = 80

F32 = jnp.float32


def _wid(cid, sid):
    return cid * NS + sid


def _make_sc_deg(n_nodes, chunks):
    mesh = plsc.VectorSubcoreMesh(core_axis_name="c", subcore_axis_name="s")

    @functools.partial(
        pl.kernel,
        out_type=jax.ShapeDtypeStruct((NW, n_nodes), F32),
        mesh=mesh,
        compiler_params=pltpu.CompilerParams(needs_layout_passes=False),
        scratch_types=[
            pltpu.VMEM((chunks, K), jnp.int32),
            pltpu.VMEM((n_nodes,), F32),
        ],
    )
    def sc_deg(row_hbm, deg_out, rbuf, degbuf):
        cid = lax.axis_index("c")
        sid = lax.axis_index("s")
        wid = _wid(cid, sid)
        pltpu.sync_copy(row_hbm.at[wid], rbuf)

        zero16 = jnp.zeros((LK,), F32)
        ones16 = jnp.ones((LK,), F32)

        def zero_body(i, _):
            degbuf[pl.ds(i * LK, LK)] = zero16
            return 0

        lax.fori_loop(0, n_nodes // LK, zero_body, 0)

        def chunk_body(c, _):
            def lane_body(j, _):
                idx = rbuf[c, pl.ds(j * LK, LK)]
                plsc.addupdate_scatter(degbuf, [idx], ones16)
                return 0

            lax.fori_loop(0, K // LK, lane_body, 0)
            return 0

        lax.fori_loop(0, chunks, chunk_body, 0)
        pltpu.sync_copy(degbuf, deg_out.at[wid])

    return sc_deg


def _make_sc_spmm_b(n_nodes, chunks, d, kk):
    mesh = plsc.VectorSubcoreMesh(core_axis_name="c", subcore_axis_name="s")
    npt = n_nodes // NS
    assert npt % kk == 0 and chunks % 2 == 0
    half = chunks // 2
    assert half % 2 == 0 and half % 8 == 0

    @functools.partial(
        pl.kernel,
        out_type=jax.ShapeDtypeStruct((NC, n_nodes, d), F32),
        mesh=mesh,
        compiler_params=pltpu.CompilerParams(needs_layout_passes=False),
        scratch_types=[
            pltpu.VMEM((half, kk), jnp.int32),
            pltpu.VMEM((half, kk), jnp.int32),
            pltpu.VMEM((kk, d), F32),
            pltpu.VMEM((kk, d), F32),
            pltpu.VMEM_SHARED((n_nodes, d), F32),
            pltpu.SemaphoreType.DMA,
            pltpu.SemaphoreType.DMA,
        ],
    )
    def sc_spmm(row_hbm, col_hbm, xs_hbm, zt_out,
                rbuf, cbuf, gbuf0, gbuf1, zsh, sem0, sem1):
        cid = lax.axis_index("c")
        sid = lax.axis_index("s")
        wid = _wid(cid, sid)

        zero16 = jnp.zeros((LK,), F32)

        def zero_zb(i, _):
            def zlane(j, _):
                gbuf0[i, pl.ds(j * LK, LK)] = zero16
                return 0
            lax.fori_loop(0, d // LK, zlane, 0)
            return 0

        lax.fori_loop(0, kk, zero_zb, 0)

        def zero_stripe(k_, _):
            pltpu.sync_copy(gbuf0, zsh.at[pl.ds(sid * npt + k_ * kk, kk)])
            return 0

        lax.fori_loop(0, npt // kk, zero_stripe, 0)
        plsc.subcore_barrier()

        def g_issue(c, buf, sem_):
            pltpu.async_copy(xs_hbm.at[cbuf.at[c]], buf, sem_)

        def g_wait(c, buf, sem_):
            pltpu.make_async_copy(xs_hbm.at[cbuf.at[c]], buf, sem_).wait()

        def s_add(c, buf):
            pltpu.sync_copy(buf, zsh.at[rbuf.at[c]], add=True)

        def do_half(hf, _):
            pltpu.sync_copy(row_hbm.at[wid].at[pl.ds(hf * half, half)], rbuf)
            pltpu.sync_copy(col_hbm.at[wid].at[pl.ds(hf * half, half)], cbuf)
            g_issue(0, gbuf0, sem0)

            def pair(i, _):
                c0 = 2 * i
                g_issue(c0 + 1, gbuf1, sem1)
                g_wait(c0, gbuf0, sem0)
                s_add(c0, gbuf0)

                @pl.when(c0 + 2 < half)
                def _():
                    g_issue(c0 + 2, gbuf0, sem0)

                g_wait(c0 + 1, gbuf1, sem1)
                s_add(c0 + 1, gbuf1)
                return 0

            lax.fori_loop(0, half // 2, pair, 0)
            return 0

        lax.fori_loop(0, 2, do_half, 0)
        plsc.subcore_barrier()

        pltpu.sync_copy(zsh.at[pl.ds(sid * npt, npt)],
                        zt_out.at[cid].at[pl.ds(sid * npt, npt)])

    return sc_spmm


def _make_sc_st(n_nodes, chunks):
    mesh = plsc.VectorSubcoreMesh(core_axis_name="c", subcore_axis_name="s")

    @functools.partial(
        pl.kernel,
        out_type=jax.ShapeDtypeStruct((NW, n_nodes), F32),
        mesh=mesh,
        compiler_params=pltpu.CompilerParams(needs_layout_passes=False),
        scratch_types=[
            pltpu.VMEM((chunks, K), jnp.int32),
            pltpu.VMEM((chunks, K), jnp.int32),
            pltpu.VMEM((n_nodes,), F32),
            pltpu.VMEM((n_nodes,), F32),
        ],
    )
    def sc_st(row_hbm, col_hbm, dis_hbm, st_out, rbuf, cbuf, disbuf, sbuf):
        cid = lax.axis_index("c")
        sid = lax.axis_index("s")
        wid = _wid(cid, sid)
        pltpu.sync_copy(row_hbm.at[wid], rbuf)
        pltpu.sync_copy(col_hbm.at[wid], cbuf)
        pltpu.sync_copy(dis_hbm, disbuf)

        zero16 = jnp.zeros((LK,), F32)

        def zero_s(i, _):
            sbuf[pl.ds(i * LK, LK)] = zero16
            return 0

        lax.fori_loop(0, n_nodes // LK, zero_s, 0)

        def chunk_body(c, _):
            def lane_body(j, _):
                cidx = cbuf[c, pl.ds(j * LK, LK)]
                vals = plsc.load_gather(disbuf, [cidx])
                ridx = rbuf[c, pl.ds(j * LK, LK)]
                plsc.addupdate_scatter(sbuf, [ridx], vals)
                return 0

            lax.fori_loop(0, K // LK, lane_body, 0)
            return 0

        lax.fori_loop(0, chunks, chunk_body, 0)
        pltpu.sync_copy(sbuf, st_out.at[wid])

    return sc_st


def _make_sc_spmm_c(n_nodes, chunks, d, slabs, kk):
    mesh = plsc.VectorSubcoreMesh(core_axis_name="c", subcore_axis_name="s")
    npt = n_nodes // NS
    assert npt % kk == 0 and chunks % 2 == 0
    half = chunks // 2
    assert half % 2 == 0 and half % 8 == 0

    @functools.partial(
        pl.kernel,
        out_type=jax.ShapeDtypeStruct((NC, slabs, n_nodes, d), F32),
        mesh=mesh,
        compiler_params=pltpu.CompilerParams(needs_layout_passes=False),
        scratch_types=[
            pltpu.VMEM((half, kk), jnp.int32),
            pltpu.VMEM((half, kk), jnp.int32),
            pltpu.VMEM((kk, d), F32),
            pltpu.VMEM((kk, d), F32),
            pltpu.VMEM_SHARED((n_nodes, d), F32),
            pltpu.SemaphoreType.DMA,
            pltpu.SemaphoreType.DMA,
        ],
    )
    def sc_spmm_c(row_hbm, col_hbm, y_hbm, o_out,
                  rbuf, cbuf, gbuf0, gbuf1, zsh, sem0, sem1):
        cid = lax.axis_index("c")
        sid = lax.axis_index("s")
        wid = _wid(cid, sid)

        zero16 = jnp.zeros((LK,), F32)

        def zero_zb(i, _):
            def zlane(j, _):
                gbuf0[i, pl.ds(j * LK, LK)] = zero16
                return 0
            lax.fori_loop(0, d // LK, zlane, 0)
            return 0

        lax.fori_loop(0, kk, zero_zb, 0)

        def zero_stripe(k_, _):
            pltpu.sync_copy(gbuf0, zsh.at[pl.ds(sid * npt + k_ * kk, kk)])
            return 0

        lax.fori_loop(0, npt // kk, zero_stripe, 0)
        plsc.subcore_barrier()

        def slab_body(s, _):
            ysl = y_hbm.at[s]

            def g_issue(c, buf, sem_):
                pltpu.async_copy(ysl.at[cbuf.at[c]], buf, sem_)

            def g_wait(c, buf, sem_):
                pltpu.make_async_copy(ysl.at[cbuf.at[c]], buf, sem_).wait()

            def s_add(c, buf):
                pltpu.sync_copy(buf, zsh.at[rbuf.at[c]], add=True)

            def do_half(hf, _):
                pltpu.sync_copy(row_hbm.at[wid].at[pl.ds(hf * half, half)], rbuf)
                pltpu.sync_copy(col_hbm.at[wid].at[pl.ds(hf * half, half)], cbuf)
                g_issue(0, gbuf0, sem0)

                def pair(i, _):
                    c0 = 2 * i
                    g_issue(c0 + 1, gbuf1, sem1)
                    g_wait(c0, gbuf0, sem0)
                    s_add(c0, gbuf0)

                    @pl.when(c0 + 2 < half)
                    def _():
                        g_issue(c0 + 2, gbuf0, sem0)

                    g_wait(c0 + 1, gbuf1, sem1)
                    s_add(c0 + 1, gbuf1)
                    return 0

                lax.fori_loop(0, half // 2, pair, 0)
                return 0

            lax.fori_loop(0, 2, do_half, 0)
            plsc.subcore_barrier()
            pltpu.sync_copy(zsh.at[pl.ds(sid * npt, npt)],
                            o_out.at[cid].at[s].at[pl.ds(sid * npt, npt)])
            plsc.subcore_barrier()
            return 0

        lax.fori_loop(0, slabs, slab_body, 0)

    return sc_spmm_c


def _tc1_body(degp_ref, x_ref, dis_ref, xs_ref):
    n, d = x_ref.shape
    npad = xs_ref.shape[0]
    deg = jnp.sum(degp_ref[...], axis=0)
    dis = jnp.where(deg > 0.0, lax.rsqrt(deg), 0.0)
    dis_ref[...] = dis[:, None]
    xs_ref[0:n, :] = dis[0:n, None] * x_ref[...]
    xs_ref[n:npad, :] = jnp.zeros((npad - n, d), F32)


def _tc2_body(zt_ref, st_ref, dis_ref, w1_ref, b1_ref, w2_ref, b2_ref, y_ref):
    nb = dis_ref.shape[0]
    i = pl.program_id(0)
    dis = dis_ref[...]
    z = (zt_ref[0] + zt_ref[1]) * dis
    st = st_ref[:, pl.ds(i * nb, nb)]
    s = jnp.sum(st, axis=0)[:, None] * dis
    h1 = jnp.maximum(
        jnp.dot(z, w1_ref[...], preferred_element_type=F32)
        + s * b1_ref[...], 0.0)
    slabs = y_ref.shape[0]
    d = y_ref.shape[2]
    for k in range(slabs):
        yk = (jnp.dot(h1, w2_ref[:, k * d:(k + 1) * d],
                      preferred_element_type=F32)
              + b2_ref[:, k * d:(k + 1) * d]) * dis
        y_ref[k] = yk


def _tc3_body(o_ref, dis_ref, out_ref):
    dis = dis_ref[...]
    cum = o_ref[0] + o_ref[1]
    slabs = cum.shape[0]
    d = cum.shape[2]
    for k in range(slabs):
        acc = cum[k] if k == 0 else cum[k] - cum[k - 1]
        out_ref[:, k * d:(k + 1) * d] = jnp.maximum(acc * dis, 0.0)


def kernel(x, edge_index, W1, b1, W2, b2):
    n, d = x.shape
    h = W1.shape[1]
    e = edge_index.shape[1]
    slabs = h // d
    chunks = e // (NW * K)
    assert e == NW * chunks * K
    npad = -(-n // (NS * 128)) * (NS * 128)
    assert n % 8 == 0

    kk = 128
    epad = -(-e // (NW * kk * 4)) * (NW * kk * 4)
    chunks2 = epad // (NW * kk)
    assert npad > n or epad == e
    pad_edge = n + (jnp.arange(epad - e, dtype=jnp.int32) % (npad - n))

    row = edge_index[0].reshape(NW, chunks, K)
    col = edge_index[1].reshape(NW, chunks, K)
    row2 = jnp.concatenate([edge_index[0], pad_edge]).reshape(NW, chunks2, kk)
    col2 = jnp.concatenate([edge_index[1], pad_edge]).reshape(NW, chunks2, kk)

    degp = _make_sc_deg(npad, chunks)(row)

    dis2, xs = pl.pallas_call(
        _tc1_body,
        out_shape=[
            jax.ShapeDtypeStruct((npad, 1), F32),
            jax.ShapeDtypeStruct((npad, d), F32),
        ],
    )(degp, x)

    zt = _make_sc_spmm_b(npad, chunks2, d, kk)(row2, col2, xs)
    st = _make_sc_st(npad, chunks)(row, col, dis2.reshape(npad))

    nb = 1024
    grid = npad // nb
    y = pl.pallas_call(
        _tc2_body,
        grid=(grid,),
        in_specs=[
            pl.BlockSpec((NC, nb, d), lambda i: (0, i, 0)),
            pl.BlockSpec((NW, npad), lambda i: (0, 0)),
            pl.BlockSpec((nb, 1), lambda i: (i, 0)),
            pl.BlockSpec((d, h), lambda i: (0, 0)),
            pl.BlockSpec((1, h), lambda i: (0, 0)),
            pl.BlockSpec((h, h), lambda i: (0, 0)),
            pl.BlockSpec((1, h), lambda i: (0, 0)),
        ],
        out_specs=pl.BlockSpec((slabs, nb, d), lambda i: (0, i, 0)),
        out_shape=jax.ShapeDtypeStruct((slabs, npad, d), F32),
    )(zt, st, dis2, W1, b1.reshape(1, h), W2, b2.reshape(1, h))

    o = _make_sc_spmm_c(npad, chunks2, d, slabs, kk)(row2, col2, y)

    nb3 = 1000
    grid3 = n // nb3
    out = pl.pallas_call(
        _tc3_body,
        grid=(grid3,),
        in_specs=[
            pl.BlockSpec((NC, slabs, nb3, d), lambda i: (0, 0, i, 0)),
            pl.BlockSpec((nb3, 1), lambda i: (i, 0)),
        ],
        out_specs=pl.BlockSpec((nb3, h), lambda i: (i, 0)),
        out_shape=jax.ShapeDtypeStruct((n, h), F32),
    )(o, dis2)

    return out
